# baseline (speedup 1.0000x reference)
import jax
import jax.numpy as jnp
import numpy as np
from jax.experimental import pallas as pl
from jax.experimental.pallas import tpu as pltpu

_DEPTH = 8
_R = 2 ** _DEPTH
_N = _R - 1


def _route_matrices() -> tuple[np.ndarray, np.ndarray]:
    a = np.zeros((2, _N, _R), dtype=np.float32)
    for r in range(_R):
        node = 0
        for i in range(_DEPTH):
            bit = (r >> (_DEPTH - 1 - i)) & 1
            a[bit, node, r] = 1.0
            node = node * 2 + 1 + bit
    return a[0], a[1]

_A0, _A1 = _route_matrices()


def _dtr_kernel(x_ref, w_ref, b_ref, a0_ref, a1_ref, out_ref):
    z = jnp.dot(x_ref[...], w_ref[...],
                preferred_element_type=jnp.float32) + b_ref[...]
    u = jnp.log1p(jnp.exp(-jnp.abs(z)))
    s = (jnp.dot(u + jnp.maximum(-z, 0.0), a0_ref[...],
                 preferred_element_type=jnp.float32)
         + jnp.dot(u + jnp.maximum(z, 0.0), a1_ref[...],
                   preferred_element_type=jnp.float32))
    out_ref[...] = jnp.exp(-s)


@jax.jit
def kernel(x, W, b):
    B, D = x.shape
    n_nodes = W.shape[1]
    tb = min(1024, B)
    b2 = b.reshape(1, n_nodes)
    a0, a1 = jnp.asarray(_A0), jnp.asarray(_A1)
    return pl.pallas_call(
        _dtr_kernel,
        grid=(B // tb,),
        in_specs=[
            pl.BlockSpec((tb, D), lambda i: (i, 0)),
            pl.BlockSpec((D, n_nodes), lambda i: (0, 0)),
            pl.BlockSpec((1, n_nodes), lambda i: (0, 0)),
            pl.BlockSpec((_N, _R), lambda i: (0, 0)),
            pl.BlockSpec((_N, _R), lambda i: (0, 0)),
        ],
        out_specs=pl.BlockSpec((tb, _R), lambda i: (i, 0)),
        out_shape=jax.ShapeDtypeStruct((B, _R), jnp.float32),
        compiler_params=pltpu.CompilerParams(
            dimension_semantics=("arbitrary",)),
    )(x, W, b2, a0, a1)


# speedup vs baseline: 1.0169x; 1.0169x over previous
import jax
import jax.numpy as jnp
import numpy as np
from jax.experimental import pallas as pl
from jax.experimental.pallas import tpu as pltpu

_DEPTH = 8
_R = 2 ** _DEPTH
_N = _R - 1


def _route_matrices() -> tuple[np.ndarray, np.ndarray]:
    a = np.zeros((2, _N, _R), dtype=np.float32)
    for r in range(_R):
        node = 0
        for i in range(_DEPTH):
            bit = (r >> (_DEPTH - 1 - i)) & 1
            a[bit, node, r] = 1.0
            node = node * 2 + 1 + bit
    return a[0], a[1]

_A0, _A1 = _route_matrices()


def _dtr_kernel(x_ref, w_ref, b_ref, a0_ref, a1_ref, out_ref):
    z = jnp.dot(x_ref[...], w_ref[...],
                preferred_element_type=jnp.float32) + b_ref[...]
    u = jnp.log1p(jnp.exp(-jnp.abs(z)))
    s = (jnp.dot(u + jnp.maximum(-z, 0.0), a0_ref[...],
                 preferred_element_type=jnp.float32)
         + jnp.dot(u + jnp.maximum(z, 0.0), a1_ref[...],
                   preferred_element_type=jnp.float32))
    out_ref[...] = jnp.exp(-s)


@jax.jit
def kernel(x, W, b):
    B, D = x.shape
    n_nodes = W.shape[1]
    tb = min(2048, B)
    b2 = b.reshape(1, n_nodes)
    a0, a1 = jnp.asarray(_A0), jnp.asarray(_A1)
    return pl.pallas_call(
        _dtr_kernel,
        grid=(B // tb,),
        in_specs=[
            pl.BlockSpec((tb, D), lambda i: (i, 0)),
            pl.BlockSpec((D, n_nodes), lambda i: (0, 0)),
            pl.BlockSpec((1, n_nodes), lambda i: (0, 0)),
            pl.BlockSpec((_N, _R), lambda i: (0, 0)),
            pl.BlockSpec((_N, _R), lambda i: (0, 0)),
        ],
        out_specs=pl.BlockSpec((tb, _R), lambda i: (i, 0)),
        out_shape=jax.ShapeDtypeStruct((B, _R), jnp.float32),
        compiler_params=pltpu.CompilerParams(
            dimension_semantics=("parallel",)),
    )(x, W, b2, a0, a1)
